# baseline (speedup 1.0000x reference)
import functools

import jax
import jax.numpy as jnp
from jax import lax
from jax.experimental import pallas as pl
from jax.experimental.pallas import tpu as pltpu
from jax.experimental.pallas import tpu_sc as plsc

_N = 10000
_E = 320000
_D = 128
_HD = 64
_NC = 2
_NS = 16
_CH = 128
_NCH = 80
_EP = _NC * _NS * _NCH * _CH
_TRASH = 10016
_NACC = 10240
_RPT = _NACC // _NS
_DEGPAD = 10240
_BLKH = 1000
_GRID = _N // (2 * _BLKH)
_DEPTH = 5

_SC_PARAMS = pltpu.CompilerParams(use_tc_tiling_on_sc=False)


def _deg_body(dst_hbm, out_hbm, idx_v, ones_v, zb_v, acc_sh):
    c = lax.axis_index("c")
    s = lax.axis_index("s")
    pltpu.sync_copy(dst_hbm.at[c].at[s], idx_v)
    ones16 = jnp.ones((16,), jnp.float32)
    zeros16 = jnp.zeros((16,), jnp.float32)
    for i in range(8):
        ones_v[pl.ds(i * 16, 16)] = ones16

    def zfill(i, carry):
        zb_v[pl.ds(i * 16, 16)] = zeros16
        return carry

    lax.fori_loop(0, 40, zfill, 0)
    pltpu.sync_copy(zb_v, acc_sh.at[pl.ds(s * 640, 640)])
    plsc.subcore_barrier()

    def body(j, carry):
        pltpu.sync_copy(ones_v, acc_sh.at[idx_v.at[j]], add=True)
        return carry

    lax.fori_loop(0, _NCH, body, 0)
    plsc.subcore_barrier()
    pltpu.sync_copy(acc_sh.at[pl.ds(s * 640, 640)],
                    out_hbm.at[pl.ds(c * _DEGPAD + s * 640, 640)])


def _spmm_body(g_hbm, src_hbm, dst_hbm, out_hbm, srcv, dstv, *rest):
    bufs = rest[:_DEPTH]
    zrow_v = rest[_DEPTH]
    acc_sh = rest[_DEPTH + 1]
    sems = rest[_DEPTH + 2:]
    c = lax.axis_index("c")
    s = lax.axis_index("s")
    pltpu.sync_copy(src_hbm.at[c].at[s], srcv)
    pltpu.sync_copy(dst_hbm.at[c].at[s], dstv)
    zeros16 = jnp.zeros((16,), jnp.float32)

    def zfill(r, carry):
        for k in range(4):
            zrow_v[r, pl.ds(k * 16, 16)] = zeros16
        return carry

    lax.fori_loop(0, 128, zfill, 0)

    for h in range(2):
        for k in range(5):
            pltpu.sync_copy(zrow_v, acc_sh.at[pl.ds(s * _RPT + k * 128, 128)])
        plsc.subcore_barrier()

        for u in range(_DEPTH):
            pltpu.async_copy(g_hbm.at[h].at[srcv.at[u]], bufs[u], sems[u])

        def body(jj, carry):
            j = jj * _DEPTH
            for u in range(_DEPTH):
                pltpu.make_async_copy(g_hbm.at[h].at[srcv.at[j + u]],
                                      bufs[u], sems[u]).wait()
                pltpu.sync_copy(bufs[u], acc_sh.at[dstv.at[j + u]], add=True)

                @pl.when(j + u + _DEPTH < _NCH)
                def _():
                    pltpu.async_copy(g_hbm.at[h].at[srcv.at[j + u + _DEPTH]],
                                     bufs[u], sems[u])

            return carry

        lax.fori_loop(0, _NCH // _DEPTH, body, 0)
        plsc.subcore_barrier()
        pltpu.sync_copy(acc_sh.at[pl.ds(s * _RPT, _RPT)],
                        out_hbm.at[c].at[h].at[pl.ds(s * _RPT, _RPT)])


@functools.lru_cache(maxsize=None)
def _deg_call():
    mesh = plsc.VectorSubcoreMesh(core_axis_name="c", subcore_axis_name="s")
    return pl.kernel(
        _deg_body,
        out_type=jax.ShapeDtypeStruct((_NC * _DEGPAD,), jnp.float32),
        mesh=mesh,
        compiler_params=_SC_PARAMS,
        scratch_types=[
            pltpu.VMEM((_NCH, _CH), jnp.int32),
            pltpu.VMEM((128,), jnp.float32),
            pltpu.VMEM((640,), jnp.float32),
            pltpu.VMEM_SHARED((_DEGPAD,), jnp.float32),
        ],
    )


@functools.lru_cache(maxsize=None)
def _spmm_call():
    mesh = plsc.VectorSubcoreMesh(core_axis_name="c", subcore_axis_name="s")
    return pl.kernel(
        _spmm_body,
        out_type=jax.ShapeDtypeStruct((_NC, 2, _NACC, _HD), jnp.float32),
        mesh=mesh,
        compiler_params=_SC_PARAMS,
        scratch_types=[
            pltpu.VMEM((_NCH, _CH), jnp.int32),
            pltpu.VMEM((_NCH, _CH), jnp.int32),
        ] + [pltpu.VMEM((_CH, _HD), jnp.float32) for _ in range(_DEPTH)] + [
            pltpu.VMEM((128, _HD), jnp.float32),
            pltpu.VMEM_SHARED((_NACC, _HD), jnp.float32),
        ] + [pltpu.SemaphoreType.DMA for _ in range(_DEPTH)],
    )


def _pack_scaled(re, ro, dp):
    ge = dp[:, 0:1] * re
    go = dp[:, _HD:_HD + 1] * ro
    return (jnp.concatenate([ge[:, :_HD], go[:, :_HD]], axis=1),
            jnp.concatenate([ge[:, _HD:], go[:, _HD:]], axis=1))


def _tc1_body(xe_ref, xo_ref, w_ref, dp_ref, g_ref):
    dp = dp_ref[...]
    re = jnp.dot(xe_ref[...], w_ref[...], preferred_element_type=jnp.float32)
    ro = jnp.dot(xo_ref[...], w_ref[...], preferred_element_type=jnp.float32)
    g0, g1 = _pack_scaled(re, ro, dp)
    g_ref[0, :, :] = g0
    g_ref[1, :, :] = g1


def _ypack(p_ref, g_ref, dp, b_ref):
    y0 = dp * (p_ref[0, 0] + p_ref[1, 0] + g_ref[0]) + b_ref[0]
    y1 = dp * (p_ref[0, 1] + p_ref[1, 1] + g_ref[1]) + b_ref[1]
    return y0, y1


def _tc_mid_body(p_ref, g_ref, dp_ref, b_ref, w_ref, gout_ref):
    dp = dp_ref[...]
    y0, y1 = _ypack(p_ref, g_ref, dp, b_ref)
    wt = w_ref[:_HD, :]
    wb = w_ref[_HD:, :]
    re = (jnp.dot(y0[:, :_HD], wt, preferred_element_type=jnp.float32)
          + jnp.dot(y1[:, :_HD], wb, preferred_element_type=jnp.float32))
    ro = (jnp.dot(y0[:, _HD:], wt, preferred_element_type=jnp.float32)
          + jnp.dot(y1[:, _HD:], wb, preferred_element_type=jnp.float32))
    g0, g1 = _pack_scaled(re, ro, dp)
    gout_ref[0, :, :] = g0
    gout_ref[1, :, :] = g1


def _tc_out_body(p_ref, g_ref, dp_ref, b_ref, out_ref):
    dp = dp_ref[...]
    y0, y1 = _ypack(p_ref, g_ref, dp, b_ref)
    out_ref[0, :, :] = jnp.concatenate([y0[:, :_HD], y1[:, :_HD]], axis=1)
    out_ref[1, :, :] = jnp.concatenate([y0[:, _HD:], y1[:, _HD:]], axis=1)


def _gspec():
    return pl.BlockSpec((2, _BLKH, _D), lambda i: (0, i, 0))


def _pspec():
    return pl.BlockSpec((_NC, 2, _BLKH, _D), lambda i: (0, 0, i, 0))


def _dpspec():
    return pl.BlockSpec((_BLKH, _D), lambda i: (i, 0))


def _tc1(xe, xo, w, dpack):
    return pl.pallas_call(
        _tc1_body,
        grid=(_GRID,),
        in_specs=[
            _dpspec(),
            _dpspec(),
            pl.BlockSpec((_D, _D), lambda i: (0, 0)),
            _dpspec(),
        ],
        out_specs=_gspec(),
        out_shape=jax.ShapeDtypeStruct((2, _N // 2, _D), jnp.float32),
    )(xe, xo, w, dpack)


def _tc_mid(p, g, dpack, bpack, w):
    return pl.pallas_call(
        _tc_mid_body,
        grid=(_GRID,),
        in_specs=[
            _pspec(),
            _gspec(),
            _dpspec(),
            pl.BlockSpec((2, 1, _D), lambda i: (0, 0, 0)),
            pl.BlockSpec((_D, _D), lambda i: (0, 0)),
        ],
        out_specs=_gspec(),
        out_shape=jax.ShapeDtypeStruct((2, _N // 2, _D), jnp.float32),
    )(p, g, dpack, bpack, w)


def _tc_out(p, g, dpack, bpack):
    return pl.pallas_call(
        _tc_out_body,
        grid=(_GRID,),
        in_specs=[
            _pspec(),
            _gspec(),
            _dpspec(),
            pl.BlockSpec((2, 1, _D), lambda i: (0, 0, 0)),
        ],
        out_specs=_gspec(),
        out_shape=jax.ShapeDtypeStruct((2, _N // 2, _D), jnp.float32),
    )(p, g, dpack, bpack)


def kernel(x, edge_index, W1, b1, W2, b2, W3, b3):
    src = jnp.pad(edge_index[0], (0, _EP - _E)).reshape(_NC, _NS, _NCH, _CH)
    dst = jnp.pad(edge_index[1], (0, _EP - _E),
                  constant_values=_TRASH).reshape(_NC, _NS, _NCH, _CH)

    degp = _deg_call()(dst)
    dinv = lax.rsqrt(degp[:_N] + degp[_DEGPAD:_DEGPAD + _N] + 1.0)
    dpack = jnp.repeat(dinv, _HD).reshape(_N // 2, _D)

    xe = x[0::2]
    xo = x[1::2]

    def bpack(b):
        return jnp.concatenate([jnp.tile(b[:_HD], 2),
                                jnp.tile(b[_HD:], 2)]).reshape(2, 1, _D)

    g = _tc1(xe, xo, W1, dpack)
    p = _spmm_call()(g.reshape(2, _N, _HD), src, dst)
    g = _tc_mid(p.reshape(_NC, 2, _NACC // 2, _D), g, dpack, bpack(b1), W2)
    p = _spmm_call()(g.reshape(2, _N, _HD), src, dst)
    g = _tc_mid(p.reshape(_NC, 2, _NACC // 2, _D), g, dpack, bpack(b2), W3)
    p = _spmm_call()(g.reshape(2, _N, _HD), src, dst)
    eo = _tc_out(p.reshape(_NC, 2, _NACC // 2, _D), g, dpack, bpack(b3))
    return jnp.stack([eo[0], eo[1]], axis=1).reshape(_N, _D)
